# baseline (speedup 1.0000x reference)
import functools

import jax
import jax.numpy as jnp
from jax import lax
from jax.experimental import pallas as pl
from jax.experimental.pallas import tpu as pltpu
from jax.experimental.pallas import tpu_sc as plsc

NFEAT = 128
MAX_SIZE = 100
NUM_STAMPS = 8
LANES = 16

NUM_CORES = 2
NUM_SUBCORES = 16
NUM_WORKERS = NUM_CORES * NUM_SUBCORES

ROWS_PER_CHUNK = 16
GATHER_ROWS = ROWS_PER_CHUNK * NUM_STAMPS
NBUF = 4


def _sc_kernel(n_rows, x2d_hbm, table_hbm, out_hbm,
               fidx_v, rows_v, acc_v, gsems, osems):
    rows_per_worker = n_rows // NUM_WORKERS
    chunks = rows_per_worker // ROWS_PER_CHUNK
    wid = lax.axis_index("s") * NUM_CORES + lax.axis_index("c")
    base_row = wid * rows_per_worker

    pltpu.sync_copy(x2d_hbm.at[pl.ds(wid * chunks, chunks)], fidx_v)
    pat = (lax.iota(jnp.int32, LANES) % NUM_STAMPS) * MAX_SIZE

    @pl.loop(0, chunks)
    def _(r):
        for g in range(GATHER_ROWS // LANES):
            sl = pl.ds(g * LANES, LANES)
            fidx_v[r, sl] = fidx_v[r, sl] + pat

    def start_gather(ch, b):
        pltpu.async_copy(table_hbm.at[fidx_v.at[ch]], rows_v.at[b], gsems[b])

    def wait_gather(ch, b):
        pltpu.make_async_copy(table_hbm.at[fidx_v.at[ch]], rows_v.at[b],
                              gsems[b]).wait()

    def compute(b):
        @plsc.parallel_loop(0, ROWS_PER_CHUNK, unroll=2)
        def _(n):
            for f in range(NFEAT // LANES):
                sl = pl.ds(f * LANES, LANES)
                v = [rows_v[b, n * NUM_STAMPS + i, sl] for i in range(NUM_STAMPS)]
                while len(v) > 1:
                    v = [v[i] + v[i + 1] for i in range(0, len(v), 2)]
                acc_v[b, n, sl] = v[0]

    def start_out(ch, b):
        r0 = base_row + ch * ROWS_PER_CHUNK
        pltpu.async_copy(acc_v.at[b], out_hbm.at[pl.ds(r0, ROWS_PER_CHUNK)],
                         osems[b])

    def wait_out(ch, b):
        r0 = base_row + ch * ROWS_PER_CHUNK
        pltpu.make_async_copy(acc_v.at[b],
                              out_hbm.at[pl.ds(r0, ROWS_PER_CHUNK)],
                              osems[b]).wait()

    for b in range(NBUF):
        start_gather(b, b)

    @pl.loop(0, chunks - NBUF, step=NBUF)
    def _(c):
        for b in range(NBUF):
            ch = c + b

            @pl.when(ch >= NBUF)
            def _():
                wait_out(ch - NBUF, b)

            wait_gather(ch, b)
            compute(b)
            start_out(ch, b)
            start_gather(ch + NBUF, b)

    for b in range(NBUF):
        ch = chunks - NBUF + b
        if ch >= NBUF:
            wait_out(ch - NBUF, b)
        wait_gather(ch, b)
        compute(b)
        start_out(ch, b)
    for b in range(NBUF):
        wait_out(chunks - NBUF + b, b)


def kernel(x, tables):
    b, l, num_stamps = x.shape
    n_rows = b * l
    x2d = jnp.asarray(x, jnp.int32).reshape(
        n_rows * num_stamps // GATHER_ROWS, GATHER_ROWS)
    tab2d = tables.reshape(tables.shape[0] * tables.shape[1], tables.shape[2])
    chunks = n_rows // NUM_WORKERS // ROWS_PER_CHUNK

    mesh = plsc.VectorSubcoreMesh(core_axis_name="c", subcore_axis_name="s")
    run = pl.kernel(
        functools.partial(_sc_kernel, n_rows),
        out_type=jax.ShapeDtypeStruct((n_rows, NFEAT), tables.dtype),
        mesh=mesh,
        scratch_types=[
            pltpu.VMEM((chunks, GATHER_ROWS), jnp.int32),
            pltpu.VMEM((NBUF, GATHER_ROWS, NFEAT), jnp.float32),
            pltpu.VMEM((NBUF, ROWS_PER_CHUNK, NFEAT), jnp.float32),
            [pltpu.SemaphoreType.DMA] * NBUF,
            [pltpu.SemaphoreType.DMA] * NBUF,
        ],
    )
    out = run(x2d, tab2d)
    return out.reshape(b, l, NFEAT)


# speedup vs baseline: 1.1644x; 1.1644x over previous
import dataclasses
import functools

import jax
import jax.numpy as jnp
from jax import lax
from jax.experimental import pallas as pl
from jax.experimental.pallas import tpu as pltpu
from jax.experimental.pallas import tpu_sc as plsc

NFEAT = 128
MAX_SIZE = 100
NUM_STAMPS = 8
LANES = 16

NUM_CORES = 2
NUM_SUBCORES = 16
NUM_WORKERS = NUM_CORES * NUM_SUBCORES

ROWS_PER_CHUNK = 16
GATHER_ROWS = ROWS_PER_CHUNK * NUM_STAMPS
NBUF = 4


def _sc_kernel(n_rows, x2d_hbm, table_hbm, out_hbm,
               fidx_v, rows_v, acc_v, gsems, osems):
    rows_per_worker = n_rows // NUM_WORKERS
    chunks = rows_per_worker // ROWS_PER_CHUNK
    wid = lax.axis_index("s") * NUM_CORES + lax.axis_index("c")
    base_row = wid * rows_per_worker

    pltpu.sync_copy(x2d_hbm.at[pl.ds(wid * chunks, chunks)], fidx_v)
    pat = (lax.iota(jnp.int32, LANES) % NUM_STAMPS) * MAX_SIZE

    @pl.loop(0, chunks)
    def _(r):
        for g in range(GATHER_ROWS // LANES):
            sl = pl.ds(g * LANES, LANES)
            fidx_v[r, sl] = fidx_v[r, sl] + pat

    def start_gather(ch, b):
        pltpu.async_copy(table_hbm.at[fidx_v.at[ch]], rows_v.at[b], gsems[b])

    def wait_gather(ch, b):
        pltpu.make_async_copy(table_hbm.at[fidx_v.at[ch]], rows_v.at[b],
                              gsems[b]).wait()

    def compute(b):
        @plsc.parallel_loop(0, ROWS_PER_CHUNK, unroll=2)
        def _(n):
            for f in range(NFEAT // (2 * LANES)):
                sl = pl.ds(f * LANES, LANES)
                lo = []
                hi = []
                for i in range(NUM_STAMPS):
                    w = plsc.bitcast(rows_v[b, n * NUM_STAMPS + i, sl],
                                     jnp.bfloat16)
                    a, c = plsc.unpack(w, format=plsc.PackFormat.INTERLEAVED)
                    lo.append(a)
                    hi.append(c)
                for v, off in ((lo, 0), (hi, LANES)):
                    while len(v) > 1:
                        v = [v[i] + v[i + 1] for i in range(0, len(v), 2)]
                    acc_v[b, n, pl.ds(f * 2 * LANES + off, LANES)] = v[0]

    def start_out(ch, b):
        r0 = base_row + ch * ROWS_PER_CHUNK
        pltpu.async_copy(acc_v.at[b], out_hbm.at[pl.ds(r0, ROWS_PER_CHUNK)],
                         osems[b])

    def wait_out(ch, b):
        r0 = base_row + ch * ROWS_PER_CHUNK
        pltpu.make_async_copy(acc_v.at[b],
                              out_hbm.at[pl.ds(r0, ROWS_PER_CHUNK)],
                              osems[b]).wait()

    for b in range(NBUF):
        start_gather(b, b)

    @pl.loop(0, chunks - NBUF, step=NBUF)
    def _(c):
        for b in range(NBUF):
            ch = c + b

            @pl.when(ch >= NBUF)
            def _():
                wait_out(ch - NBUF, b)

            wait_gather(ch, b)
            compute(b)
            start_out(ch, b)
            start_gather(ch + NBUF, b)

    for b in range(NBUF):
        ch = chunks - NBUF + b
        if ch >= NBUF:
            wait_out(ch - NBUF, b)
        wait_gather(ch, b)
        compute(b)
        start_out(ch, b)
    for b in range(NBUF):
        wait_out(chunks - NBUF + b, b)


def kernel(x, tables):
    b, l, num_stamps = x.shape
    n_rows = b * l
    x2d = jnp.asarray(x, jnp.int32).reshape(
        n_rows * num_stamps // GATHER_ROWS, GATHER_ROWS)
    tab2d = tables.reshape(tables.shape[0] * tables.shape[1], tables.shape[2])
    perm = (jnp.arange(NFEAT) // 2) + (jnp.arange(NFEAT) % 2) * LANES
    perm = perm + (jnp.arange(NFEAT) // (2 * LANES)) * LANES
    tab2d = tab2d[:, perm].astype(jnp.bfloat16)
    tab2d = jax.lax.bitcast_convert_type(
        tab2d.reshape(tab2d.shape[0], NFEAT // 2, 2), jnp.int32)
    chunks = n_rows // NUM_WORKERS // ROWS_PER_CHUNK

    mesh = plsc.VectorSubcoreMesh(core_axis_name="c", subcore_axis_name="s")
    cp = pltpu.CompilerParams()
    if "needs_layout_passes" in pltpu.CompilerParams.__dataclass_fields__:
        cp = dataclasses.replace(cp, needs_layout_passes=False)
    if "use_tc_tiling_on_sc" in pltpu.CompilerParams.__dataclass_fields__:
        cp = dataclasses.replace(cp, use_tc_tiling_on_sc=False)
    run = pl.kernel(
        functools.partial(_sc_kernel, n_rows),
        out_type=jax.ShapeDtypeStruct((n_rows, NFEAT), tables.dtype),
        mesh=mesh,
        scratch_types=[
            pltpu.VMEM((chunks, GATHER_ROWS), jnp.int32),
            pltpu.VMEM((NBUF, GATHER_ROWS, NFEAT // 2), jnp.int32),
            pltpu.VMEM((NBUF, ROWS_PER_CHUNK, NFEAT), jnp.float32),
            [pltpu.SemaphoreType.DMA] * NBUF,
            [pltpu.SemaphoreType.DMA] * NBUF,
        ],
        compiler_params=cp,
    )
    out = run(x2d, tab2d)
    return out.reshape(b, l, NFEAT)


# speedup vs baseline: 1.4348x; 1.2323x over previous
import dataclasses
import functools

import jax
import jax.numpy as jnp
from jax import lax
from jax.experimental import pallas as pl
from jax.experimental.pallas import tpu as pltpu
from jax.experimental.pallas import tpu_sc as plsc

NFEAT = 128
MAX_SIZE = 100
NUM_STAMPS = 8
NUM_PAIRS = NUM_STAMPS // 2
PAIR_SIZE = MAX_SIZE * MAX_SIZE
LANES = 16

NUM_CORES = 2
NUM_SUBCORES = 16
NUM_WORKERS = NUM_CORES * NUM_SUBCORES

ROWS_PER_CHUNK = 32
GATHER_ROWS = ROWS_PER_CHUNK * NUM_PAIRS
RAW_COLS = 128
NBUF = 4


def _pair_table_kernel(t_ref, o_ref):
    a = t_ref[0]
    b = t_ref[1]
    o_ref[...] = (a[:, None, :] + b[None, :, :])[None]


def _build_pair_tables(tables):
    return pl.pallas_call(
        _pair_table_kernel,
        grid=(NUM_PAIRS,),
        in_specs=[pl.BlockSpec((2, MAX_SIZE, NFEAT), lambda p: (p, 0, 0))],
        out_specs=pl.BlockSpec((1, MAX_SIZE, MAX_SIZE, NFEAT),
                               lambda p: (p, 0, 0, 0)),
        out_shape=jax.ShapeDtypeStruct(
            (NUM_PAIRS, MAX_SIZE, MAX_SIZE, NFEAT), tables.dtype),
    )(tables[:NUM_STAMPS])


def _sc_kernel(n_rows, x2d_hbm, ptab_hbm, out_hbm,
               raw_v, cidx_v, rows_v, acc_v, gsems, osems):
    rows_per_worker = n_rows // NUM_WORKERS
    chunks = rows_per_worker // ROWS_PER_CHUNK
    raw_rows = rows_per_worker * NUM_STAMPS // RAW_COLS
    wid = lax.axis_index("s") * NUM_CORES + lax.axis_index("c")
    base_row = wid * rows_per_worker

    pltpu.sync_copy(x2d_hbm.at[pl.ds(wid * raw_rows, raw_rows)], raw_v)

    iota = lax.iota(jnp.int32, LANES)
    pbase = (iota % NUM_PAIRS) * PAIR_SIZE
    groups = rows_per_worker * NUM_PAIRS // LANES

    @pl.loop(0, groups)
    def _(g):
        qa = g * 2 * LANES + 2 * iota
        qb = qa + 1
        a = plsc.load_gather(raw_v, [qa >> 7, qa & 127])
        b = plsc.load_gather(raw_v, [qb >> 7, qb & 127])
        v = a * MAX_SIZE + b + pbase
        cidx_v[g >> 3, pl.ds((g & 7) * LANES, LANES)] = v

    def start_gather(ch, b):
        pltpu.async_copy(ptab_hbm.at[cidx_v.at[ch]], rows_v.at[b], gsems[b])

    def wait_gather(ch, b):
        pltpu.make_async_copy(ptab_hbm.at[cidx_v.at[ch]], rows_v.at[b],
                              gsems[b]).wait()

    def compute(b):
        @plsc.parallel_loop(0, ROWS_PER_CHUNK, unroll=2)
        def _(n):
            for f in range(NFEAT // (2 * LANES)):
                sl = pl.ds(f * LANES, LANES)
                lo = []
                hi = []
                for i in range(NUM_PAIRS):
                    w = plsc.bitcast(rows_v[b, n * NUM_PAIRS + i, sl],
                                     jnp.bfloat16)
                    u, v = plsc.unpack(w, format=plsc.PackFormat.INTERLEAVED)
                    lo.append(u)
                    hi.append(v)
                for v, off in ((lo, 0), (hi, LANES)):
                    while len(v) > 1:
                        v = [v[i] + v[i + 1] for i in range(0, len(v), 2)]
                    acc_v[b, n, pl.ds(f * 2 * LANES + off, LANES)] = v[0]

    def start_out(ch, b):
        r0 = base_row + ch * ROWS_PER_CHUNK
        pltpu.async_copy(acc_v.at[b], out_hbm.at[pl.ds(r0, ROWS_PER_CHUNK)],
                         osems[b])

    def wait_out(ch, b):
        r0 = base_row + ch * ROWS_PER_CHUNK
        pltpu.make_async_copy(acc_v.at[b],
                              out_hbm.at[pl.ds(r0, ROWS_PER_CHUNK)],
                              osems[b]).wait()

    for b in range(NBUF):
        start_gather(b, b)

    @pl.loop(0, chunks - NBUF, step=NBUF)
    def _(c):
        for b in range(NBUF):
            ch = c + b

            @pl.when(ch >= NBUF)
            def _():
                wait_out(ch - NBUF, b)

            wait_gather(ch, b)
            compute(b)
            start_out(ch, b)
            start_gather(ch + NBUF, b)

    for b in range(NBUF):
        ch = chunks - NBUF + b
        if ch >= NBUF:
            wait_out(ch - NBUF, b)
        wait_gather(ch, b)
        compute(b)
        start_out(ch, b)
    for b in range(NBUF):
        wait_out(chunks - NBUF + b, b)


def kernel(x, tables):
    b, l, num_stamps = x.shape
    n_rows = b * l
    x2d = jnp.asarray(x, jnp.int32).reshape(
        n_rows * num_stamps // RAW_COLS, RAW_COLS)

    ptab = _build_pair_tables(tables)
    ptab = ptab.reshape(NUM_PAIRS * PAIR_SIZE, NFEAT)
    perm = (jnp.arange(NFEAT) // 2) + (jnp.arange(NFEAT) % 2) * LANES
    perm = perm + (jnp.arange(NFEAT) // (2 * LANES)) * LANES
    ptab = ptab[:, perm].astype(jnp.bfloat16)
    ptab = jax.lax.bitcast_convert_type(
        ptab.reshape(ptab.shape[0], NFEAT // 2, 2), jnp.int32)

    chunks = n_rows // NUM_WORKERS // ROWS_PER_CHUNK
    raw_rows = n_rows * num_stamps // NUM_WORKERS // RAW_COLS

    mesh = plsc.VectorSubcoreMesh(core_axis_name="c", subcore_axis_name="s")
    cp = pltpu.CompilerParams()
    if "needs_layout_passes" in pltpu.CompilerParams.__dataclass_fields__:
        cp = dataclasses.replace(cp, needs_layout_passes=False)
    if "use_tc_tiling_on_sc" in pltpu.CompilerParams.__dataclass_fields__:
        cp = dataclasses.replace(cp, use_tc_tiling_on_sc=False)
    run = pl.kernel(
        functools.partial(_sc_kernel, n_rows),
        out_type=jax.ShapeDtypeStruct((n_rows, NFEAT), tables.dtype),
        mesh=mesh,
        scratch_types=[
            pltpu.VMEM((raw_rows, RAW_COLS), jnp.int32),
            pltpu.VMEM((chunks, GATHER_ROWS), jnp.int32),
            pltpu.VMEM((NBUF, GATHER_ROWS, NFEAT // 2), jnp.int32),
            pltpu.VMEM((NBUF, ROWS_PER_CHUNK, NFEAT), jnp.float32),
            [pltpu.SemaphoreType.DMA] * NBUF,
            [pltpu.SemaphoreType.DMA] * NBUF,
        ],
        compiler_params=cp,
    )
    out = run(x2d, ptab)
    return out.reshape(b, l, NFEAT)


# speedup vs baseline: 1.5182x; 1.0582x over previous
import dataclasses
import functools

import jax
import jax.numpy as jnp
from jax import lax
from jax.experimental import pallas as pl
from jax.experimental.pallas import tpu as pltpu
from jax.experimental.pallas import tpu_sc as plsc

NFEAT = 128
MAX_SIZE = 100
NUM_STAMPS = 8
NUM_PAIRS = NUM_STAMPS // 2
PAIR_SIZE = MAX_SIZE * MAX_SIZE
LANES = 16

NUM_CORES = 2
NUM_SUBCORES = 16
NUM_WORKERS = NUM_CORES * NUM_SUBCORES

ROWS_PER_CHUNK = 25
GATHER_ROWS = ROWS_PER_CHUNK * NUM_PAIRS
NBUF = 4


def _pair_table_kernel(t_ref, o_ref):
    a = t_ref[0]
    b = t_ref[1]
    o_ref[...] = (a[:, None, :] + b[None, :, :]).astype(jnp.bfloat16)[None]


def _build_pair_tables(tables):
    return pl.pallas_call(
        _pair_table_kernel,
        grid=(NUM_PAIRS,),
        in_specs=[pl.BlockSpec((2, MAX_SIZE, NFEAT), lambda p: (p, 0, 0))],
        out_specs=pl.BlockSpec((1, MAX_SIZE, MAX_SIZE, NFEAT),
                               lambda p: (p, 0, 0, 0)),
        out_shape=jax.ShapeDtypeStruct(
            (NUM_PAIRS, MAX_SIZE, MAX_SIZE, NFEAT), jnp.bfloat16),
    )(tables)


def _sc_kernel(bsz, seq, x_hbm, ptab_hbm, out_hbm,
               raw_v, cidx_v, rows_v, acc_v, gsems, osems):
    n_rows = bsz * seq
    rows_per_worker = n_rows // NUM_WORKERS
    b_per_worker = rows_per_worker // seq
    chunks = rows_per_worker // ROWS_PER_CHUNK
    wid = lax.axis_index("s") * NUM_CORES + lax.axis_index("c")

    pltpu.sync_copy(x_hbm.at[pl.ds(wid * b_per_worker, b_per_worker)], raw_v)

    iota = lax.iota(jnp.int32, LANES)
    pbase = (iota % NUM_PAIRS) * PAIR_SIZE
    groups = rows_per_worker * NUM_PAIRS // LANES
    raw_minor = seq * NUM_STAMPS

    @pl.loop(0, groups)
    def _(g):
        qa = g * 2 * LANES + 2 * iota
        qb = qa + 1
        ra = qa // raw_minor
        rem_a = qa - ra * raw_minor
        rb = qb // raw_minor
        rem_b = qb - rb * raw_minor
        a = plsc.load_gather(raw_v, [ra, rem_a >> 3, rem_a & 7])
        b = plsc.load_gather(raw_v, [rb, rem_b >> 3, rem_b & 7])
        v = a * MAX_SIZE + b + pbase
        q = g * LANES + iota
        plsc.store_scatter(cidx_v, [q // GATHER_ROWS, q % GATHER_ROWS], v)

    def start_gather(ch, b):
        pltpu.async_copy(ptab_hbm.at[cidx_v.at[ch]], rows_v.at[b], gsems[b])

    def wait_gather(ch, b):
        pltpu.make_async_copy(ptab_hbm.at[cidx_v.at[ch]], rows_v.at[b],
                              gsems[b]).wait()

    def compute(b):
        @plsc.parallel_loop(0, ROWS_PER_CHUNK, unroll=2)
        def _(n):
            for f in range(NFEAT // (2 * LANES)):
                sl = pl.ds(f * LANES, LANES)
                lo = []
                hi = []
                for i in range(NUM_PAIRS):
                    w = plsc.bitcast(rows_v[b, n * NUM_PAIRS + i, sl],
                                     jnp.bfloat16)
                    u, v = plsc.unpack(w, format=plsc.PackFormat.INTERLEAVED)
                    lo.append(u)
                    hi.append(v)
                for v, off in ((lo, 0), (hi, LANES)):
                    while len(v) > 1:
                        v = [v[i] + v[i + 1] for i in range(0, len(v), 2)]
                    acc_v[b, n, pl.ds(f * 2 * LANES + off, LANES)] = v[0]

    def out_view(ch):
        r0 = wid * rows_per_worker + ch * ROWS_PER_CHUNK
        return out_hbm.at[r0 // seq, pl.ds(r0 % seq, ROWS_PER_CHUNK)]

    def start_out(ch, b):
        pltpu.async_copy(acc_v.at[b], out_view(ch), osems[b])

    def wait_out(ch, b):
        pltpu.make_async_copy(acc_v.at[b], out_view(ch), osems[b]).wait()

    for b in range(NBUF):
        start_gather(b, b)

    @pl.loop(0, chunks - NBUF, step=NBUF)
    def _(c):
        for b in range(NBUF):
            ch = c + b

            @pl.when(ch >= NBUF)
            def _():
                wait_out(ch - NBUF, b)

            wait_gather(ch, b)
            compute(b)
            start_out(ch, b)
            start_gather(ch + NBUF, b)

    for b in range(NBUF):
        ch = chunks - NBUF + b
        if ch >= NBUF:
            wait_out(ch - NBUF, b)
        wait_gather(ch, b)
        compute(b)
        start_out(ch, b)
    for b in range(NBUF):
        wait_out(chunks - NBUF + b, b)


def kernel(x, tables):
    bsz, seq, num_stamps = x.shape
    xi = jnp.asarray(x, jnp.int32)

    perm = (jnp.arange(NFEAT) // 2) + (jnp.arange(NFEAT) % 2) * LANES
    perm = perm + (jnp.arange(NFEAT) // (2 * LANES)) * LANES
    tabs = tables[:NUM_STAMPS][:, :, perm]

    ptab = _build_pair_tables(tabs)
    ptab = jax.lax.bitcast_convert_type(
        ptab.reshape(NUM_PAIRS * PAIR_SIZE, NFEAT // 2, 2), jnp.int32)

    n_rows = bsz * seq
    chunks = n_rows // NUM_WORKERS // ROWS_PER_CHUNK
    b_per_worker = bsz // NUM_WORKERS

    mesh = plsc.VectorSubcoreMesh(core_axis_name="c", subcore_axis_name="s")
    cp = pltpu.CompilerParams()
    if "needs_layout_passes" in pltpu.CompilerParams.__dataclass_fields__:
        cp = dataclasses.replace(cp, needs_layout_passes=False)
    if "use_tc_tiling_on_sc" in pltpu.CompilerParams.__dataclass_fields__:
        cp = dataclasses.replace(cp, use_tc_tiling_on_sc=False)
    run = pl.kernel(
        functools.partial(_sc_kernel, bsz, seq),
        out_type=jax.ShapeDtypeStruct((bsz, seq, NFEAT), tables.dtype),
        mesh=mesh,
        scratch_types=[
            pltpu.VMEM((b_per_worker, seq, NUM_STAMPS), jnp.int32),
            pltpu.VMEM((chunks, GATHER_ROWS), jnp.int32),
            pltpu.VMEM((NBUF, GATHER_ROWS, NFEAT // 2), jnp.int32),
            pltpu.VMEM((NBUF, ROWS_PER_CHUNK, NFEAT), jnp.float32),
            [pltpu.SemaphoreType.DMA] * NBUF,
            [pltpu.SemaphoreType.DMA] * NBUF,
        ],
        compiler_params=cp,
    )
    return run(xi, ptab)


# speedup vs baseline: 1.6731x; 1.1020x over previous
import dataclasses
import functools

import jax
import jax.numpy as jnp
from jax import lax
from jax.experimental import pallas as pl
from jax.experimental.pallas import tpu as pltpu
from jax.experimental.pallas import tpu_sc as plsc

NFEAT = 128
MAX_SIZE = 100
NUM_STAMPS = 8
NUM_PAIRS = NUM_STAMPS // 2
PAIR_SIZE = MAX_SIZE * MAX_SIZE
LANES = 16

NUM_CORES = 2
NUM_SUBCORES = 16
NUM_WORKERS = NUM_CORES * NUM_SUBCORES

ROWS_PER_CHUNK = 25
GATHER_ROWS = ROWS_PER_CHUNK * NUM_PAIRS
NBUF = 2


def _pair_table_kernel(t_ref, o_ref):
    a = t_ref[0]
    b = t_ref[1]
    o_ref[...] = (a[:, None, :] + b[None, :, :])[None]


def _build_pair_tables(tables):
    return pl.pallas_call(
        _pair_table_kernel,
        grid=(NUM_PAIRS,),
        in_specs=[pl.BlockSpec((2, MAX_SIZE, NFEAT), lambda p: (p, 0, 0))],
        out_specs=pl.BlockSpec((1, MAX_SIZE, MAX_SIZE, NFEAT),
                               lambda p: (p, 0, 0, 0)),
        out_shape=jax.ShapeDtypeStruct(
            (NUM_PAIRS, MAX_SIZE, MAX_SIZE, NFEAT), tables.dtype),
    )(tables[:NUM_STAMPS])


def _sc_kernel(bsz, seq, x_hbm, ptab_hbm, out_hbm,
               raw_v, cidx_v, rows_v, acc_v, gsems, osems):
    n_rows = bsz * seq
    rows_per_worker = n_rows // NUM_WORKERS
    b_per_worker = rows_per_worker // seq
    chunks = rows_per_worker // ROWS_PER_CHUNK
    wid = lax.axis_index("s") * NUM_CORES + lax.axis_index("c")

    pltpu.sync_copy(x_hbm.at[pl.ds(wid * b_per_worker, b_per_worker)], raw_v)

    iota = lax.iota(jnp.int32, LANES)
    pbase = (iota % NUM_PAIRS) * PAIR_SIZE
    groups = rows_per_worker * NUM_PAIRS // LANES
    raw_minor = seq * NUM_STAMPS

    @pl.loop(0, groups)
    def _(g):
        qa = g * 2 * LANES + 2 * iota
        qb = qa + 1
        ra = qa // raw_minor
        rem_a = qa - ra * raw_minor
        rb = qb // raw_minor
        rem_b = qb - rb * raw_minor
        a = plsc.load_gather(raw_v, [ra, rem_a >> 3, rem_a & 7])
        b = plsc.load_gather(raw_v, [rb, rem_b >> 3, rem_b & 7])
        v = a * MAX_SIZE + b + pbase
        q = g * LANES + iota
        plsc.store_scatter(cidx_v, [q // GATHER_ROWS, q % GATHER_ROWS], v)

    def start_gather(ch, b):
        pltpu.async_copy(ptab_hbm.at[cidx_v.at[ch]], rows_v.at[b], gsems[b])

    def wait_gather(ch, b):
        pltpu.make_async_copy(ptab_hbm.at[cidx_v.at[ch]], rows_v.at[b],
                              gsems[b]).wait()

    def compute(b):
        @plsc.parallel_loop(0, ROWS_PER_CHUNK, unroll=2)
        def _(n):
            for f in range(NFEAT // LANES):
                sl = pl.ds(f * LANES, LANES)
                v = [rows_v[b, n * NUM_PAIRS + i, sl]
                     for i in range(NUM_PAIRS)]
                while len(v) > 1:
                    v = [v[i] + v[i + 1] for i in range(0, len(v), 2)]
                acc_v[b, n, sl] = v[0]

    def out_view(ch):
        r0 = wid * rows_per_worker + ch * ROWS_PER_CHUNK
        return out_hbm.at[r0 // seq, pl.ds(r0 % seq, ROWS_PER_CHUNK)]

    def start_out(ch, b):
        pltpu.async_copy(acc_v.at[b], out_view(ch), osems[b])

    def wait_out(ch, b):
        pltpu.make_async_copy(acc_v.at[b], out_view(ch), osems[b]).wait()

    for b in range(NBUF):
        start_gather(b, b)

    @pl.loop(0, chunks - NBUF, step=NBUF)
    def _(c):
        for b in range(NBUF):
            ch = c + b

            @pl.when(ch >= NBUF)
            def _():
                wait_out(ch - NBUF, b)

            wait_gather(ch, b)
            compute(b)
            start_out(ch, b)
            start_gather(ch + NBUF, b)

    for b in range(NBUF):
        ch = chunks - NBUF + b
        if ch >= NBUF:
            wait_out(ch - NBUF, b)
        wait_gather(ch, b)
        compute(b)
        start_out(ch, b)
    for b in range(NBUF):
        wait_out(chunks - NBUF + b, b)


def kernel(x, tables):
    bsz, seq, num_stamps = x.shape
    xi = jnp.asarray(x, jnp.int32)

    ptab = _build_pair_tables(tables)
    ptab = ptab.reshape(NUM_PAIRS * PAIR_SIZE, NFEAT)

    n_rows = bsz * seq
    chunks = n_rows // NUM_WORKERS // ROWS_PER_CHUNK
    b_per_worker = bsz // NUM_WORKERS

    mesh = plsc.VectorSubcoreMesh(core_axis_name="c", subcore_axis_name="s")
    cp = pltpu.CompilerParams()
    if "needs_layout_passes" in pltpu.CompilerParams.__dataclass_fields__:
        cp = dataclasses.replace(cp, needs_layout_passes=False)
    if "use_tc_tiling_on_sc" in pltpu.CompilerParams.__dataclass_fields__:
        cp = dataclasses.replace(cp, use_tc_tiling_on_sc=False)
    run = pl.kernel(
        functools.partial(_sc_kernel, bsz, seq),
        out_type=jax.ShapeDtypeStruct((bsz, seq, NFEAT), tables.dtype),
        mesh=mesh,
        scratch_types=[
            pltpu.VMEM((b_per_worker, seq, NUM_STAMPS), jnp.int32),
            pltpu.VMEM((chunks, GATHER_ROWS), jnp.int32),
            pltpu.VMEM((NBUF, GATHER_ROWS, NFEAT), jnp.float32),
            pltpu.VMEM((NBUF, ROWS_PER_CHUNK, NFEAT), jnp.float32),
            [pltpu.SemaphoreType.DMA] * NBUF,
            [pltpu.SemaphoreType.DMA] * NBUF,
        ],
        compiler_params=cp,
    )
    return run(xi, ptab)


# speedup vs baseline: 2.1317x; 1.2741x over previous
import dataclasses
import functools

import jax
import jax.numpy as jnp
from jax import lax
from jax.experimental import pallas as pl
from jax.experimental.pallas import tpu as pltpu
from jax.experimental.pallas import tpu_sc as plsc

NFEAT = 128
MAX_SIZE = 100
NUM_STAMPS = 8
NUM_PAIRS = NUM_STAMPS // 2
PAIR_SIZE = MAX_SIZE * MAX_SIZE
LANES = 16

NUM_CORES = 2
NUM_SUBCORES = 16
NUM_WORKERS = NUM_CORES * NUM_SUBCORES

ROWS_PER_CHUNK = 32
GATHER_ROWS = ROWS_PER_CHUNK * NUM_PAIRS
RAW_COLS = 128
SEQ = 50
SEQ_PAD = 56
NBUF = 2
NACC = 4


def _pair_table_kernel(t_ref, o_ref):
    a = t_ref[0]
    b = t_ref[1]
    o_ref[...] = (a[:, None, :] + b[None, :, :])[None]


def _build_pair_tables(tables):
    return pl.pallas_call(
        _pair_table_kernel,
        grid=(NUM_PAIRS,),
        in_specs=[pl.BlockSpec((2, MAX_SIZE, NFEAT), lambda p: (p, 0, 0))],
        out_specs=pl.BlockSpec((1, MAX_SIZE, MAX_SIZE, NFEAT),
                               lambda p: (p, 0, 0, 0)),
        out_shape=jax.ShapeDtypeStruct(
            (NUM_PAIRS, MAX_SIZE, MAX_SIZE, NFEAT), tables.dtype),
    )(tables[:NUM_STAMPS])


def _sc_kernel(bsz, seq, x2d_hbm, ptab_hbm, out_hbm,
               raw_v, rows_v, acc_v, gsems, osems):
    n_rows = bsz * seq
    rows_per_worker = n_rows // NUM_WORKERS
    e_per_worker = rows_per_worker // seq
    chunks = rows_per_worker // ROWS_PER_CHUNK
    raw_rows = rows_per_worker * NUM_STAMPS // RAW_COLS
    wid = lax.axis_index("s") * NUM_CORES + lax.axis_index("c")
    base_e = wid * e_per_worker

    pltpu.sync_copy(x2d_hbm.at[pl.ds(wid * raw_rows, raw_rows)], raw_v)

    iota = lax.iota(jnp.int32, LANES)
    pbase = (iota % NUM_PAIRS) * PAIR_SIZE
    groups = rows_per_worker * NUM_PAIRS // LANES

    @pl.loop(0, groups)
    def _(g):
        qa = g * 2 * LANES + 2 * iota
        qb = qa + 1
        a = plsc.load_gather(raw_v, [qa >> 7, qa & 127])
        b = plsc.load_gather(raw_v, [qb >> 7, qb & 127])
        v = a * MAX_SIZE + b + pbase
        q = g * LANES + iota
        plsc.store_scatter(raw_v, [q >> 7, q & 127], v)

    def start_gather(ch, b):
        pltpu.async_copy(ptab_hbm.at[raw_v.at[ch]], rows_v.at[b], gsems[b])

    def wait_gather(ch, b):
        pltpu.make_async_copy(ptab_hbm.at[raw_v.at[ch]], rows_v.at[b],
                              gsems[b]).wait()

    def compute(ch, b):
        @plsc.parallel_loop(0, ROWS_PER_CHUNK, unroll=2)
        def _(n):
            g0 = ch * ROWS_PER_CHUNK + n
            e = g0 // seq
            l = g0 - e * seq
            slot = e & (NACC - 1)
            for f in range(NFEAT // LANES):
                sl = pl.ds(f * LANES, LANES)
                v = [rows_v[b, n * NUM_PAIRS + i, sl]
                     for i in range(NUM_PAIRS)]
                while len(v) > 1:
                    v = [v[i] + v[i + 1] for i in range(0, len(v), 2)]
                acc_v[slot, l, sl] = v[0]

    def start_out(e, s):
        pltpu.async_copy(acc_v.at[e & (NACC - 1)],
                         out_hbm.at[pl.ds((base_e + e) * SEQ_PAD, SEQ_PAD)],
                         osems[s])

    def wait_out(e, s):
        pltpu.make_async_copy(acc_v.at[e & (NACC - 1)],
                              out_hbm.at[pl.ds((base_e + e) * SEQ_PAD,
                                               SEQ_PAD)],
                              osems[s]).wait()

    def entry_flow(ch):
        if isinstance(ch, int):
            s_new = (ch * ROWS_PER_CHUNK + ROWS_PER_CHUNK - 1) // seq
            s_old = (ch * ROWS_PER_CHUNK - 1) // seq if ch else -1
            if s_new > s_old and s_new >= NACC:
                wait_out(s_new - NACC, (s_new - NACC) & (NACC - 1))
        else:
            s_new = (ch * ROWS_PER_CHUNK + ROWS_PER_CHUNK - 1) // seq
            s_old = (ch * ROWS_PER_CHUNK - 1) // seq
            began = jnp.logical_and(s_new > s_old, s_new >= NACC)
            for s in range(NACC):
                @pl.when(jnp.logical_and(
                    began, ((s_new - NACC) & (NACC - 1)) == s))
                def _():
                    wait_out(s_new - NACC, s)

    def fire_flow(ch):
        if isinstance(ch, int):
            ndone = (ch * ROWS_PER_CHUNK + ROWS_PER_CHUNK) // seq
            nprev = (ch * ROWS_PER_CHUNK) // seq
            if ndone > nprev:
                start_out(ndone - 1, (ndone - 1) & (NACC - 1))
        else:
            ndone = (ch * ROWS_PER_CHUNK + ROWS_PER_CHUNK) // seq
            nprev = (ch * ROWS_PER_CHUNK) // seq
            fire = ndone > nprev
            for s in range(NACC):
                @pl.when(jnp.logical_and(fire, ((ndone - 1) & (NACC - 1)) == s))
                def _():
                    start_out(ndone - 1, s)

    for b in range(NBUF):
        start_gather(b, b)

    @pl.loop(0, chunks - NBUF, step=NBUF)
    def _(c):
        for b in range(NBUF):
            ch = c + b
            entry_flow(ch)
            wait_gather(ch, b)
            compute(ch, b)
            fire_flow(ch)
            start_gather(ch + NBUF, b)

    for b in range(NBUF):
        ch = chunks - NBUF + b
        entry_flow(ch)
        wait_gather(ch, b)
        compute(ch, b)
        fire_flow(ch)
    for e in range(e_per_worker - NACC, e_per_worker):
        wait_out(e, e & (NACC - 1))


def kernel(x, tables):
    bsz, seq, num_stamps = x.shape
    n_rows = bsz * seq
    x2d = jnp.asarray(x, jnp.int32).reshape(
        n_rows * num_stamps // RAW_COLS, RAW_COLS)

    ptab = _build_pair_tables(tables)
    ptab = ptab.reshape(NUM_PAIRS * PAIR_SIZE, NFEAT)

    raw_rows = n_rows * num_stamps // NUM_WORKERS // RAW_COLS

    mesh = plsc.VectorSubcoreMesh(core_axis_name="c", subcore_axis_name="s")
    cp = pltpu.CompilerParams()
    if "needs_layout_passes" in pltpu.CompilerParams.__dataclass_fields__:
        cp = dataclasses.replace(cp, needs_layout_passes=False)
    run = pl.kernel(
        functools.partial(_sc_kernel, bsz, seq),
        out_type=jax.ShapeDtypeStruct((bsz * SEQ_PAD, NFEAT), tables.dtype),
        mesh=mesh,
        scratch_types=[
            pltpu.VMEM((raw_rows, RAW_COLS), jnp.int32),
            pltpu.VMEM((NBUF, GATHER_ROWS, NFEAT), jnp.float32),
            pltpu.VMEM((NACC, SEQ_PAD, NFEAT), jnp.float32),
            [pltpu.SemaphoreType.DMA] * NBUF,
            [pltpu.SemaphoreType.DMA] * NACC,
        ],
        compiler_params=cp,
    )
    out = run(x2d, ptab).reshape(bsz, SEQ_PAD, NFEAT)
    return out[:, :seq, :]


# speedup vs baseline: 2.2308x; 1.0465x over previous
import dataclasses
import functools

import jax
import jax.numpy as jnp
from jax import lax
from jax.experimental import pallas as pl
from jax.experimental.pallas import tpu as pltpu
from jax.experimental.pallas import tpu_sc as plsc

NFEAT = 128
MAX_SIZE = 100
NUM_STAMPS = 8
NUM_PAIRS = NUM_STAMPS // 2
PAIR_PAD = 104
PAIR_SIZE = MAX_SIZE * PAIR_PAD
LANES = 16

NUM_CORES = 2
NUM_SUBCORES = 16
NUM_WORKERS = NUM_CORES * NUM_SUBCORES

ROWS_PER_CHUNK = 32
GATHER_ROWS = ROWS_PER_CHUNK * NUM_PAIRS
RAW_COLS = 128
SEQ = 50
SEQ_PAD = 56
NBUF = 2
NACC = 4


def _pair_table_kernel(t_ref, o_ref):
    a = t_ref[0]
    b = t_ref[1]
    o_ref[0, :, pl.ds(0, MAX_SIZE), :] = a[:, None, :] + b[None, :, :]


def _build_pair_tables(tables):
    return pl.pallas_call(
        _pair_table_kernel,
        grid=(NUM_PAIRS,),
        in_specs=[pl.BlockSpec((2, MAX_SIZE, NFEAT), lambda p: (p, 0, 0))],
        out_specs=pl.BlockSpec((1, MAX_SIZE, PAIR_PAD, NFEAT),
                               lambda p: (p, 0, 0, 0)),
        out_shape=jax.ShapeDtypeStruct(
            (NUM_PAIRS, MAX_SIZE, PAIR_PAD, NFEAT), tables.dtype),
    )(tables[:NUM_STAMPS])


def _sc_kernel(bsz, seq, x2d_hbm, ptab_hbm, out_hbm,
               raw_v, rows_v, acc_v, gsems, osems):
    n_rows = bsz * seq
    rows_per_worker = n_rows // NUM_WORKERS
    e_per_worker = rows_per_worker // seq
    chunks = rows_per_worker // ROWS_PER_CHUNK
    raw_rows = rows_per_worker * NUM_STAMPS // RAW_COLS
    wid = lax.axis_index("s") * NUM_CORES + lax.axis_index("c")
    base_e = wid * e_per_worker

    pltpu.sync_copy(x2d_hbm.at[pl.ds(wid * raw_rows, raw_rows)], raw_v)

    iota = lax.iota(jnp.int32, LANES)
    pbase = (iota % NUM_PAIRS) * PAIR_SIZE
    groups = rows_per_worker * NUM_PAIRS // LANES

    @pl.loop(0, groups)
    def _(g):
        qa = g * 2 * LANES + 2 * iota
        qb = qa + 1
        a = plsc.load_gather(raw_v, [qa >> 7, qa & 127])
        b = plsc.load_gather(raw_v, [qb >> 7, qb & 127])
        v = a * PAIR_PAD + b + pbase
        q = g * LANES + iota
        plsc.store_scatter(raw_v, [q >> 7, q & 127], v)

    def start_gather(ch, b):
        pltpu.async_copy(ptab_hbm.at[raw_v.at[ch]], rows_v.at[b], gsems[b])

    def wait_gather(ch, b):
        pltpu.make_async_copy(ptab_hbm.at[raw_v.at[ch]], rows_v.at[b],
                              gsems[b]).wait()

    def compute(ch, b):
        @plsc.parallel_loop(0, ROWS_PER_CHUNK, unroll=2)
        def _(n):
            g0 = ch * ROWS_PER_CHUNK + n
            e = g0 // seq
            l = g0 - e * seq
            slot = e & (NACC - 1)
            for f in range(NFEAT // LANES):
                sl = pl.ds(f * LANES, LANES)
                v = [rows_v[b, n * NUM_PAIRS + i, sl]
                     for i in range(NUM_PAIRS)]
                while len(v) > 1:
                    v = [v[i] + v[i + 1] for i in range(0, len(v), 2)]
                acc_v[slot, l, sl] = v[0]

    def start_out(e, s):
        pltpu.async_copy(acc_v.at[e & (NACC - 1)],
                         out_hbm.at[pl.ds((base_e + e) * SEQ_PAD, SEQ_PAD)],
                         osems[s])

    def wait_out(e, s):
        pltpu.make_async_copy(acc_v.at[e & (NACC - 1)],
                              out_hbm.at[pl.ds((base_e + e) * SEQ_PAD,
                                               SEQ_PAD)],
                              osems[s]).wait()

    def entry_flow(ch):
        if isinstance(ch, int):
            s_new = (ch * ROWS_PER_CHUNK + ROWS_PER_CHUNK - 1) // seq
            s_old = (ch * ROWS_PER_CHUNK - 1) // seq if ch else -1
            if s_new > s_old and s_new >= NACC:
                wait_out(s_new - NACC, (s_new - NACC) & (NACC - 1))
        else:
            s_new = (ch * ROWS_PER_CHUNK + ROWS_PER_CHUNK - 1) // seq
            s_old = (ch * ROWS_PER_CHUNK - 1) // seq
            began = jnp.logical_and(s_new > s_old, s_new >= NACC)
            for s in range(NACC):
                @pl.when(jnp.logical_and(
                    began, ((s_new - NACC) & (NACC - 1)) == s))
                def _():
                    wait_out(s_new - NACC, s)

    def fire_flow(ch):
        if isinstance(ch, int):
            ndone = (ch * ROWS_PER_CHUNK + ROWS_PER_CHUNK) // seq
            nprev = (ch * ROWS_PER_CHUNK) // seq
            if ndone > nprev:
                start_out(ndone - 1, (ndone - 1) & (NACC - 1))
        else:
            ndone = (ch * ROWS_PER_CHUNK + ROWS_PER_CHUNK) // seq
            nprev = (ch * ROWS_PER_CHUNK) // seq
            fire = ndone > nprev
            for s in range(NACC):
                @pl.when(jnp.logical_and(fire, ((ndone - 1) & (NACC - 1)) == s))
                def _():
                    start_out(ndone - 1, s)

    for b in range(NBUF):
        start_gather(b, b)

    @pl.loop(0, chunks - NBUF, step=NBUF)
    def _(c):
        for b in range(NBUF):
            ch = c + b
            entry_flow(ch)
            wait_gather(ch, b)
            compute(ch, b)
            fire_flow(ch)
            start_gather(ch + NBUF, b)

    for b in range(NBUF):
        ch = chunks - NBUF + b
        entry_flow(ch)
        wait_gather(ch, b)
        compute(ch, b)
        fire_flow(ch)
    for e in range(e_per_worker - NACC, e_per_worker):
        wait_out(e, e & (NACC - 1))


def kernel(x, tables):
    bsz, seq, num_stamps = x.shape
    n_rows = bsz * seq
    x2d = jnp.asarray(x, jnp.int32).reshape(
        n_rows * num_stamps // RAW_COLS, RAW_COLS)

    ptab = _build_pair_tables(tables)
    ptab = ptab.reshape(NUM_PAIRS * PAIR_SIZE, NFEAT)

    raw_rows = n_rows * num_stamps // NUM_WORKERS // RAW_COLS

    mesh = plsc.VectorSubcoreMesh(core_axis_name="c", subcore_axis_name="s")
    cp = pltpu.CompilerParams()
    if "needs_layout_passes" in pltpu.CompilerParams.__dataclass_fields__:
        cp = dataclasses.replace(cp, needs_layout_passes=False)
    run = pl.kernel(
        functools.partial(_sc_kernel, bsz, seq),
        out_type=jax.ShapeDtypeStruct((bsz * SEQ_PAD, NFEAT), tables.dtype),
        mesh=mesh,
        scratch_types=[
            pltpu.VMEM((raw_rows, RAW_COLS), jnp.int32),
            pltpu.VMEM((NBUF, GATHER_ROWS, NFEAT), jnp.float32),
            pltpu.VMEM((NACC, SEQ_PAD, NFEAT), jnp.float32),
            [pltpu.SemaphoreType.DMA] * NBUF,
            [pltpu.SemaphoreType.DMA] * NACC,
        ],
        compiler_params=cp,
    )
    out = run(x2d, ptab).reshape(bsz, SEQ_PAD, NFEAT)
    return out[:, :seq, :]
